# Initial kernel scaffold; baseline (speedup 1.0000x reference)
import functools

import jax
import jax.numpy as jnp
from jax import lax
from jax.experimental import pallas as pl
from jax.experimental.pallas import tpu as pltpu
from jax.experimental.pallas import tpu_sc as plsc


NC, NS = 2, 16
NW = NC * NS


@functools.lru_cache(maxsize=None)
def _make_gather(n_idx, vocab, dim, chunk):
    assert n_idx % NW == 0
    per_w = n_idx // NW
    assert per_w % chunk == 0
    n_chunks = per_w // chunk
    mesh = plsc.VectorSubcoreMesh(core_axis_name="c", subcore_axis_name="s")

    def body(ids_hbm, table_hbm, out_hbm, idx_v, rows_v, sem_g):
        wid = lax.axis_index("s") * NC + lax.axis_index("c")
        base = wid * per_w

        def idx_copy(i, s):
            pltpu.sync_copy(ids_hbm.at[pl.ds(base + i * chunk, chunk)],
                            idx_v.at[s])

        def gather_start(s):
            return pltpu.async_copy(table_hbm.at[idx_v.at[s]],
                                    rows_v.at[s], sem_g)

        def out_copy(i, s):
            pltpu.sync_copy(rows_v.at[s],
                            out_hbm.at[pl.ds(base + i * chunk, chunk)])

        idx_copy(0, 0)
        g = gather_start(0)
        for i in range(n_chunks):
            s = i % 2
            if i + 1 < n_chunks:
                idx_copy(i + 1, 1 - s)
            g.wait()
            if i + 1 < n_chunks:
                g = gather_start(1 - s)
            out_copy(i, s)

    return pl.kernel(
        body,
        out_type=jax.ShapeDtypeStruct((n_idx, dim), jnp.float32),
        mesh=mesh,
        scratch_types=[
            pltpu.VMEM((2, chunk), jnp.int32),
            pltpu.VMEM((2, chunk, dim), jnp.float32),
            pltpu.SemaphoreType.DMA,
        ],
    )


def kernel(input_ids, table):
    batch, seq = input_ids.shape
    vocab, dim = table.shape
    ids_flat = input_ids.reshape(-1).astype(jnp.int32)
    fn = _make_gather(ids_flat.shape[0], vocab, dim, 1600)
    out = fn(ids_flat, table)
    return out.reshape(batch, seq, dim)


# speedup vs baseline: 1.4883x; 1.4883x over previous
import functools

import jax
import jax.numpy as jnp
from jax import lax
from jax.experimental import pallas as pl
from jax.experimental.pallas import tpu as pltpu
from jax.experimental.pallas import tpu_sc as plsc


NC, NS = 2, 16
NW = NC * NS


@functools.lru_cache(maxsize=None)
def _make_gather(n_idx, vocab, dim, chunk):
    assert n_idx % NW == 0
    per_w = n_idx // NW
    assert per_w % chunk == 0
    n_chunks = per_w // chunk
    mesh = plsc.VectorSubcoreMesh(core_axis_name="c", subcore_axis_name="s")

    def body(ids_hbm, table_hbm, out_hbm, idx_a, idx_b, rows_a, rows_b,
             sem_g):
        wid = lax.axis_index("s") * NC + lax.axis_index("c")
        base = wid * per_w
        idx_v = (idx_a, idx_b)
        rows_v = (rows_a, rows_b)

        def idx_copy(i, s):
            pltpu.sync_copy(ids_hbm.at[pl.ds(base + i * chunk, chunk)],
                            idx_v[s])

        def gather_start(s):
            return pltpu.async_copy(table_hbm.at[idx_v[s]],
                                    rows_v[s], sem_g)

        def out_copy(i, s):
            pltpu.sync_copy(rows_v[s],
                            out_hbm.at[pl.ds(base + i * chunk, chunk)])

        idx_copy(0, 0)
        g = gather_start(0)
        for i in range(n_chunks):
            s = i % 2
            if i + 1 < n_chunks:
                idx_copy(i + 1, 1 - s)
            g.wait()
            if i + 1 < n_chunks:
                g = gather_start(1 - s)
            out_copy(i, s)

    return pl.kernel(
        body,
        out_type=jax.ShapeDtypeStruct((n_idx, dim), jnp.float32),
        mesh=mesh,
        scratch_types=[
            pltpu.VMEM((chunk,), jnp.int32),
            pltpu.VMEM((chunk,), jnp.int32),
            pltpu.VMEM((chunk, dim), jnp.float32),
            pltpu.VMEM((chunk, dim), jnp.float32),
            pltpu.SemaphoreType.DMA,
        ],
        compiler_params=pltpu.CompilerParams(use_tc_tiling_on_sc=False),
    )


def kernel(input_ids, table):
    batch, seq = input_ids.shape
    vocab, dim = table.shape
    ids_flat = input_ids.reshape(-1).astype(jnp.int32)
    fn = _make_gather(ids_flat.shape[0], vocab, dim, 1600)
    out = fn(ids_flat, table)
    return out.reshape(batch, seq, dim)
